# Initial kernel scaffold; baseline (speedup 1.0000x reference)
import jax
import jax.numpy as jnp
from jax.experimental import pallas as pl


def kernel(x, edge_index, batch, W1, b1, W2, b2, W3, b3, Wl, bl):
    raise NotImplementedError("write your pallas kernel here")


# speedup vs baseline: 12.2275x; 12.2275x over previous
import functools

import jax
import jax.numpy as jnp
from jax import lax
from jax.experimental import pallas as pl
from jax.experimental.pallas import tpu as pltpu
from jax.experimental.pallas import tpu_sc as plsc

N = 10000
NP = 10240
E = 160000
D = 128
H1 = 9000
H1P = 9216
H2 = 6000
H2P = 6144
F = 128
NCHUNK = H2P // F
NC = 2
NS = 16
ROWS_PER_TILE = NP // NS
CH_PER_SC = NCHUNK // NC

EP = 163840
E1_BATCH = 256
E1_NB = EP // (NC * NS * E1_BATCH)
E2_BATCH = 256
E2_NB = EP // (NS * E2_BATCH)


def _sc1_body(xs_hbm, src_hbm, dst_hbm, zeros_hbm, out_hbm,
              src_v, dst_v, gbuf, acc, sem):
    cid = lax.axis_index("c")
    sid = lax.axis_index("s")
    wid = cid * NS + sid
    base = sid * ROWS_PER_TILE
    pltpu.sync_copy(zeros_hbm, acc.at[pl.ds(base, ROWS_PER_TILE)])
    plsc.subcore_barrier()

    def edge_batch(j, carry):
        pltpu.sync_copy(src_hbm.at[wid, j], src_v)
        pltpu.sync_copy(dst_hbm.at[wid, j], dst_v)
        pltpu.async_copy(xs_hbm.at[src_v], gbuf, sem).wait()
        pltpu.sync_copy(gbuf, acc.at[dst_v], add=True)
        return carry
    lax.fori_loop(0, E1_NB, edge_batch, 0)
    plsc.subcore_barrier()
    pltpu.sync_copy(acc.at[pl.ds(base, ROWS_PER_TILE)],
                    out_hbm.at[cid].at[pl.ds(base, ROWS_PER_TILE)])


@functools.cache
def _make_sc1():
    return pl.kernel(
        _sc1_body,
        out_type=jax.ShapeDtypeStruct((NC, NP, D), jnp.float32),
        mesh=plsc.VectorSubcoreMesh(core_axis_name="c", subcore_axis_name="s",
                                    num_cores=NC, num_subcores=NS),
        scratch_types=[
            pltpu.VMEM((E1_BATCH,), jnp.int32),
            pltpu.VMEM((E1_BATCH,), jnp.int32),
            pltpu.VMEM((E1_BATCH, D), jnp.float32),
            pltpu.VMEM_SHARED((NP, D), jnp.float32),
            pltpu.SemaphoreType.DMA,
        ],
    )


def _sc1(*args):
    return _make_sc1()(*args)


def _sc2_body(ms_hbm, src_hbm, dst_hbm, zeros_hbm, out_hbm,
              src_v, dst_v, gbuf, acc, sem):
    cid = lax.axis_index("c")
    sid = lax.axis_index("s")
    base = sid * ROWS_PER_TILE

    def per_chunk(t, carry):
        c = cid * CH_PER_SC + t
        pltpu.sync_copy(zeros_hbm, acc.at[pl.ds(base, ROWS_PER_TILE)])
        plsc.subcore_barrier()

        def edge_batch(j, carry2):
            pltpu.sync_copy(src_hbm.at[c, sid, j], src_v)
            pltpu.sync_copy(dst_hbm.at[sid, j], dst_v)
            pltpu.async_copy(ms_hbm.at[src_v], gbuf, sem).wait()
            pltpu.sync_copy(gbuf, acc.at[dst_v], add=True)
            return carry2
        lax.fori_loop(0, E2_NB, edge_batch, 0)
        plsc.subcore_barrier()
        pltpu.sync_copy(acc.at[pl.ds(base, ROWS_PER_TILE)],
                        out_hbm.at[c].at[pl.ds(base, ROWS_PER_TILE)])
        plsc.subcore_barrier()
        return carry
    lax.fori_loop(0, CH_PER_SC, per_chunk, 0)


@functools.cache
def _make_sc2():
    return pl.kernel(
        _sc2_body,
        out_type=jax.ShapeDtypeStruct((NCHUNK, NP, F), jnp.float32),
        mesh=plsc.VectorSubcoreMesh(core_axis_name="c", subcore_axis_name="s",
                                    num_cores=NC, num_subcores=NS),
        scratch_types=[
            pltpu.VMEM((E2_BATCH,), jnp.int32),
            pltpu.VMEM((E2_BATCH,), jnp.int32),
            pltpu.VMEM((E2_BATCH, F), jnp.float32),
            pltpu.VMEM_SHARED((NP, F), jnp.float32),
            pltpu.SemaphoreType.DMA,
        ],
    )


def _sc2(*args):
    return _make_sc2()(*args)


def _scale_body(x_ref, dis_ref, out_ref):
    out_ref[...] = x_ref[...] * dis_ref[...]


def _scale_rows(xp, disc):
    bm = 1024
    return pl.pallas_call(
        _scale_body,
        grid=(NP // bm,),
        in_specs=[
            pl.BlockSpec((bm, D), lambda i: (i, 0)),
            pl.BlockSpec((bm, 1), lambda i: (i, 0)),
        ],
        out_specs=pl.BlockSpec((bm, D), lambda i: (i, 0)),
        out_shape=jax.ShapeDtypeStruct((NP, D), jnp.float32),
    )(xp, disc)


def _mm1_body(p_ref, xs_ref, dis_ref, w1_ref, b1_ref, out_ref):
    a = ((p_ref[0] + p_ref[1] + xs_ref[...]) * dis_ref[...]).astype(jnp.bfloat16)
    r = jnp.dot(a, w1_ref[...], preferred_element_type=jnp.float32)
    out_ref[...] = jnp.maximum(r + b1_ref[...], 0.0).astype(jnp.bfloat16)


def _mm1(parts, xs, disc, w1b, b1p):
    bm, bn = 1024, 1024
    return pl.pallas_call(
        _mm1_body,
        grid=(NP // bm, H1P // bn),
        in_specs=[
            pl.BlockSpec((2, bm, D), lambda i, j: (0, i, 0)),
            pl.BlockSpec((bm, D), lambda i, j: (i, 0)),
            pl.BlockSpec((bm, 1), lambda i, j: (i, 0)),
            pl.BlockSpec((D, bn), lambda i, j: (0, j)),
            pl.BlockSpec((1, bn), lambda i, j: (0, j)),
        ],
        out_specs=pl.BlockSpec((bm, bn), lambda i, j: (i, j)),
        out_shape=jax.ShapeDtypeStruct((NP, H1P), jnp.bfloat16),
    )(parts, xs, disc, w1b, b1p)


def _mm2_body(h_ref, w_ref, dis_ref, out_ref, acc_ref, *, nk, nchunk_blk):
    k = pl.program_id(2)

    @pl.when(k == 0)
    def _():
        acc_ref[...] = jnp.zeros_like(acc_ref)

    acc_ref[...] += jnp.dot(h_ref[...], w_ref[...],
                            preferred_element_type=jnp.float32)

    @pl.when(k == nk - 1)
    def _():
        s = acc_ref[...] * dis_ref[...]
        for c in range(nchunk_blk):
            out_ref[c] = s[:, c * F:(c + 1) * F]


def _mm2(h1r, w2b, disc):
    bm, bn, bk = 1024, H2P // 2, 512
    nk = H1P // bk
    nchunk_blk = bn // F
    return pl.pallas_call(
        functools.partial(_mm2_body, nk=nk, nchunk_blk=nchunk_blk),
        grid=(NP // bm, H2P // bn, nk),
        in_specs=[
            pl.BlockSpec((bm, bk), lambda i, j, k: (i, k)),
            pl.BlockSpec((bk, bn), lambda i, j, k: (k, j)),
            pl.BlockSpec((bm, 1), lambda i, j, k: (i, 0)),
        ],
        out_specs=pl.BlockSpec((nchunk_blk, bm, F), lambda i, j, k: (j, i, 0)),
        out_shape=jax.ShapeDtypeStruct((NCHUNK, NP, F), jnp.float32),
        scratch_shapes=[pltpu.VMEM((bm, bn), jnp.float32)],
    )(h1r, w2b, disc)


def _vred_body(qh_ref, ms_ref, u_ref, dis_ref, b2_ref, out_ref):
    i = pl.program_id(1)
    h = jnp.maximum(dis_ref[...] * (qh_ref[0] + ms_ref[0]) + b2_ref[0], 0.0)
    contrib = jnp.sum(u_ref[...] * h, axis=0, keepdims=True)[None]

    @pl.when(i == 0)
    def _():
        out_ref[...] = contrib

    @pl.when(i > 0)
    def _():
        out_ref[...] += contrib


def _vred(qh, ms, uc, disc, b2r):
    bm = 2048
    return pl.pallas_call(
        _vred_body,
        grid=(NCHUNK, NP // bm),
        in_specs=[
            pl.BlockSpec((1, bm, F), lambda c, i: (c, i, 0)),
            pl.BlockSpec((1, bm, F), lambda c, i: (c, i, 0)),
            pl.BlockSpec((bm, 1), lambda c, i: (i, 0)),
            pl.BlockSpec((bm, 1), lambda c, i: (i, 0)),
            pl.BlockSpec((1, 1, F), lambda c, i: (c, 0, 0)),
        ],
        out_specs=pl.BlockSpec((1, 1, F), lambda c, i: (c, 0, 0)),
        out_shape=jax.ShapeDtypeStruct((NCHUNK, 1, F), jnp.float32),
    )(qh, ms, uc, disc, b2r)


def _lin_body(v_ref, w_ref, b_ref, out_ref):
    out_ref[...] = jnp.dot(v_ref[...], w_ref[...],
                           preferred_element_type=jnp.float32) + b_ref[...]


def _linear(v, wp, bp, bn):
    kdim = wp.shape[0]
    ndim = wp.shape[1]
    return pl.pallas_call(
        _lin_body,
        grid=(ndim // bn,),
        in_specs=[
            pl.BlockSpec((1, kdim), lambda j: (0, 0)),
            pl.BlockSpec((kdim, bn), lambda j: (0, j)),
            pl.BlockSpec((1, bn), lambda j: (0, j)),
        ],
        out_specs=pl.BlockSpec((1, bn), lambda j: (0, j)),
        out_shape=jax.ShapeDtypeStruct((1, ndim), jnp.float32),
    )(v, wp, bp)


def kernel(x, edge_index, batch, W1, b1, W2, b2, W3, b3, Wl, bl):
    f32 = jnp.float32
    src = edge_index[0]
    dst = edge_index[1]

    deg = jnp.zeros((N,), f32).at[dst].add(1.0) + 1.0
    dis = lax.rsqrt(deg)
    u = dis * (jnp.zeros((N,), f32).at[src].add(dis[dst]) + dis) * (1.0 / N)

    disp = jnp.pad(dis, (0, NP - N))
    up = jnp.pad(u, (0, NP - N))
    disc = disp[:, None]
    uc = up[:, None]

    xp = jnp.pad(x, ((0, NP - N), (0, 0)))
    w1b = jnp.pad(W1, ((0, 0), (0, H1P - H1))).astype(jnp.bfloat16)
    b1p = jnp.pad(b1, (0, H1P - H1))[None, :]
    w2b = jnp.pad(W2, ((0, H1P - H1), (0, H2P - H2))).astype(jnp.bfloat16)
    b2r = jnp.pad(b2, (0, H2P - H2)).reshape(NCHUNK, 1, F)
    w3p = jnp.pad(W3, ((0, H2P - H2), (0, 2048 - 2000)))
    b3p = jnp.pad(b3, (0, 2048 - 2000))[None, :]
    wlp = jnp.pad(Wl, ((0, 2048 - 2000), (0, 128 - 10)))
    blp = jnp.pad(bl, (0, 128 - 10))[None, :]

    srcp = jnp.pad(src, (0, EP - E), constant_values=NP - 1)
    dstp = jnp.pad(dst, (0, EP - E), constant_values=NP - 1)
    src1 = srcp.reshape(NC * NS, E1_NB, E1_BATCH)
    dst1 = dstp.reshape(NC * NS, E1_NB, E1_BATCH)
    src2 = srcp.reshape(NS, E2_NB, E2_BATCH)
    dst2 = dstp.reshape(NS, E2_NB, E2_BATCH)
    zeros_z = jnp.zeros((ROWS_PER_TILE, F), f32)

    xs = _scale_rows(xp, disc)
    parts = _sc1(xs, src1, dst1, zeros_z)
    h1r = _mm1(parts, xs, disc, w1b, b1p)

    ms = _mm2(h1r, w2b, disc)
    src2c = src2[None] + (jnp.arange(NCHUNK, dtype=jnp.int32) * NP)[:, None, None, None]
    qh = _sc2(ms.reshape(NCHUNK * NP, F), src2c, dst2, zeros_z)

    v = _vred(qh, ms, uc, disc, b2r).reshape(1, H2P)
    t = _linear(v, w3p, b3p, 512)
    out = _linear(t, wlp, blp, 128)
    return out[:, :10]


# speedup vs baseline: 13.2425x; 1.0830x over previous
import functools

import jax
import jax.numpy as jnp
from jax import lax
from jax.experimental import pallas as pl
from jax.experimental.pallas import tpu as pltpu
from jax.experimental.pallas import tpu_sc as plsc

N = 10000
NP = 10240
E = 160000
D = 128
H1 = 9000
H1P = 9216
H2 = 6000
H2P = 6144
F = 128
NCHUNK = H2P // F
NC = 2
NS = 16
ROWS_PER_TILE = NP // NS
CH_PER_SC = NCHUNK // NC

EP = 163840
E1_BATCH = 256
E1_NB = EP // (NC * NS * E1_BATCH)
E2_BATCH = 128
E2_NB = EP // (NS * E2_BATCH)
NBUF2 = 2


def _sc1_body(xs_hbm, src_hbm, dst_hbm, zeros_hbm, out_hbm,
              src_v, dst_v, gbuf, acc, sem):
    cid = lax.axis_index("c")
    sid = lax.axis_index("s")
    wid = cid * NS + sid
    base = sid * ROWS_PER_TILE
    pltpu.sync_copy(zeros_hbm, acc.at[pl.ds(base, ROWS_PER_TILE)])
    plsc.subcore_barrier()

    def edge_batch(j, carry):
        pltpu.sync_copy(src_hbm.at[wid, j], src_v)
        pltpu.sync_copy(dst_hbm.at[wid, j], dst_v)
        pltpu.async_copy(xs_hbm.at[src_v], gbuf, sem).wait()
        pltpu.sync_copy(gbuf, acc.at[dst_v], add=True)
        return carry
    lax.fori_loop(0, E1_NB, edge_batch, 0)
    plsc.subcore_barrier()
    pltpu.sync_copy(acc.at[pl.ds(base, ROWS_PER_TILE)],
                    out_hbm.at[cid].at[pl.ds(base, ROWS_PER_TILE)])


@functools.cache
def _make_sc1():
    return pl.kernel(
        _sc1_body,
        out_type=jax.ShapeDtypeStruct((NC, NP, D), jnp.float32),
        mesh=plsc.VectorSubcoreMesh(core_axis_name="c", subcore_axis_name="s",
                                    num_cores=NC, num_subcores=NS),
        scratch_types=[
            pltpu.VMEM((E1_BATCH,), jnp.int32),
            pltpu.VMEM((E1_BATCH,), jnp.int32),
            pltpu.VMEM((E1_BATCH, D), jnp.float32),
            pltpu.VMEM_SHARED((NP, D), jnp.float32),
            pltpu.SemaphoreType.DMA,
        ],
    )


def _sc1(*args):
    return _make_sc1()(*args)


def _sc2_body(ms_hbm, src_hbm, dst_hbm, zeros_hbm, out_hbm,
              srcv0, srcv1, dstv0, dstv1, gbuf0, gbuf1, acc,
              isem0, isem1, gsem0, gsem1, ssem0, ssem1):
    cid = lax.axis_index("c")
    sid = lax.axis_index("s")
    base = sid * ROWS_PER_TILE
    srcv = (srcv0, srcv1)
    dstv = (dstv0, dstv1)
    gbuf = (gbuf0, gbuf1)
    isem = (isem0, isem1)
    gsem = (gsem0, gsem1)
    ssem = (ssem0, ssem1)

    def idx_pair(c, j, b):
        return (pltpu.make_async_copy(src_hbm.at[c, sid, j], srcv[b], isem[b]),
                pltpu.make_async_copy(dst_hbm.at[sid, j], dstv[b], isem[b]))

    def gather_d(b):
        return pltpu.make_async_copy(ms_hbm.at[srcv[b]], gbuf[b], gsem[b])

    def scatter_d(b):
        return pltpu.make_async_copy(gbuf[b], acc.at[dstv[b]], ssem[b])

    ngrp = E2_NB // NBUF2

    def per_chunk(t, carry):
        c = cid * CH_PER_SC + t
        pltpu.sync_copy(zeros_hbm, acc.at[pl.ds(base, ROWS_PER_TILE)])
        plsc.subcore_barrier()
        for b in range(NBUF2):
            a, d = idx_pair(c, b, b)
            a.start()
            d.start()

        def group(g, carry2):
            j0 = g * NBUF2
            for b in range(NBUF2):
                a, d = idx_pair(c, j0 + b, b)
                a.wait()
                d.wait()
                gather_d(b).start()
            for b in range(NBUF2):
                gather_d(b).wait()
                scatter_d(b).start(add=True)
            for b in range(NBUF2):
                scatter_d(b).wait()
                jn = jnp.minimum(j0 + NBUF2 + b, E2_NB - 1)
                a, d = idx_pair(c, jn, b)
                a.start()
                d.start()
            return carry2
        lax.fori_loop(0, ngrp, group, 0)
        for b in range(NBUF2):
            a, d = idx_pair(c, E2_NB - 1, b)
            a.wait()
            d.wait()
        plsc.subcore_barrier()
        pltpu.sync_copy(acc.at[pl.ds(base, ROWS_PER_TILE)],
                        out_hbm.at[c].at[pl.ds(base, ROWS_PER_TILE)])
        plsc.subcore_barrier()
        return carry
    lax.fori_loop(0, CH_PER_SC, per_chunk, 0)


@functools.cache
def _make_sc2():
    return pl.kernel(
        _sc2_body,
        out_type=jax.ShapeDtypeStruct((NCHUNK, NP, F), jnp.float32),
        mesh=plsc.VectorSubcoreMesh(core_axis_name="c", subcore_axis_name="s",
                                    num_cores=NC, num_subcores=NS),
        scratch_types=[
            pltpu.VMEM((E2_BATCH,), jnp.int32),
            pltpu.VMEM((E2_BATCH,), jnp.int32),
            pltpu.VMEM((E2_BATCH,), jnp.int32),
            pltpu.VMEM((E2_BATCH,), jnp.int32),
            pltpu.VMEM((E2_BATCH, F), jnp.float32),
            pltpu.VMEM((E2_BATCH, F), jnp.float32),
            pltpu.VMEM_SHARED((NP, F), jnp.float32),
            pltpu.SemaphoreType.DMA,
            pltpu.SemaphoreType.DMA,
            pltpu.SemaphoreType.DMA,
            pltpu.SemaphoreType.DMA,
            pltpu.SemaphoreType.DMA,
            pltpu.SemaphoreType.DMA,
        ],
    )


def _sc2(*args):
    return _make_sc2()(*args)


def _scale_body(x_ref, dis_ref, out_ref):
    out_ref[...] = x_ref[...] * dis_ref[...]


def _scale_rows(xp, disc):
    bm = 1024
    return pl.pallas_call(
        _scale_body,
        grid=(NP // bm,),
        in_specs=[
            pl.BlockSpec((bm, D), lambda i: (i, 0)),
            pl.BlockSpec((bm, 1), lambda i: (i, 0)),
        ],
        out_specs=pl.BlockSpec((bm, D), lambda i: (i, 0)),
        out_shape=jax.ShapeDtypeStruct((NP, D), jnp.float32),
    )(xp, disc)


def _mm1_body(p_ref, xs_ref, dis_ref, w1_ref, b1_ref, out_ref):
    a = ((p_ref[0] + p_ref[1] + xs_ref[...]) * dis_ref[...]).astype(jnp.bfloat16)
    r = jnp.dot(a, w1_ref[...], preferred_element_type=jnp.float32)
    out_ref[...] = jnp.maximum(r + b1_ref[...], 0.0).astype(jnp.bfloat16)


def _mm1(parts, xs, disc, w1b, b1p):
    bm, bn = 1024, 1024
    return pl.pallas_call(
        _mm1_body,
        grid=(NP // bm, H1P // bn),
        in_specs=[
            pl.BlockSpec((2, bm, D), lambda i, j: (0, i, 0)),
            pl.BlockSpec((bm, D), lambda i, j: (i, 0)),
            pl.BlockSpec((bm, 1), lambda i, j: (i, 0)),
            pl.BlockSpec((D, bn), lambda i, j: (0, j)),
            pl.BlockSpec((1, bn), lambda i, j: (0, j)),
        ],
        out_specs=pl.BlockSpec((bm, bn), lambda i, j: (i, j)),
        out_shape=jax.ShapeDtypeStruct((NP, H1P), jnp.bfloat16),
    )(parts, xs, disc, w1b, b1p)


def _mm2_body(h_ref, w_ref, dis_ref, out_ref, acc_ref, *, nk, nchunk_blk):
    k = pl.program_id(2)

    @pl.when(k == 0)
    def _():
        acc_ref[...] = jnp.zeros_like(acc_ref)

    acc_ref[...] += jnp.dot(h_ref[...], w_ref[...],
                            preferred_element_type=jnp.float32)

    @pl.when(k == nk - 1)
    def _():
        s = acc_ref[...] * dis_ref[...]
        for c in range(nchunk_blk):
            out_ref[c] = s[:, c * F:(c + 1) * F]


def _mm2(h1r, w2b, disc):
    bm, bn, bk = 1024, H2P // 2, 512
    nk = H1P // bk
    nchunk_blk = bn // F
    return pl.pallas_call(
        functools.partial(_mm2_body, nk=nk, nchunk_blk=nchunk_blk),
        grid=(NP // bm, H2P // bn, nk),
        in_specs=[
            pl.BlockSpec((bm, bk), lambda i, j, k: (i, k)),
            pl.BlockSpec((bk, bn), lambda i, j, k: (k, j)),
            pl.BlockSpec((bm, 1), lambda i, j, k: (i, 0)),
        ],
        out_specs=pl.BlockSpec((nchunk_blk, bm, F), lambda i, j, k: (j, i, 0)),
        out_shape=jax.ShapeDtypeStruct((NCHUNK, NP, F), jnp.float32),
        scratch_shapes=[pltpu.VMEM((bm, bn), jnp.float32)],
    )(h1r, w2b, disc)


def _vred_body(qh_ref, ms_ref, u_ref, dis_ref, b2_ref, out_ref):
    i = pl.program_id(1)
    h = jnp.maximum(dis_ref[...] * (qh_ref[0] + ms_ref[0]) + b2_ref[0], 0.0)
    contrib = jnp.sum(u_ref[...] * h, axis=0, keepdims=True)[None]

    @pl.when(i == 0)
    def _():
        out_ref[...] = contrib

    @pl.when(i > 0)
    def _():
        out_ref[...] += contrib


def _vred(qh, ms, uc, disc, b2r):
    bm = 2048
    return pl.pallas_call(
        _vred_body,
        grid=(NCHUNK, NP // bm),
        in_specs=[
            pl.BlockSpec((1, bm, F), lambda c, i: (c, i, 0)),
            pl.BlockSpec((1, bm, F), lambda c, i: (c, i, 0)),
            pl.BlockSpec((bm, 1), lambda c, i: (i, 0)),
            pl.BlockSpec((bm, 1), lambda c, i: (i, 0)),
            pl.BlockSpec((1, 1, F), lambda c, i: (c, 0, 0)),
        ],
        out_specs=pl.BlockSpec((1, 1, F), lambda c, i: (c, 0, 0)),
        out_shape=jax.ShapeDtypeStruct((NCHUNK, 1, F), jnp.float32),
    )(qh, ms, uc, disc, b2r)


def _lin_body(v_ref, w_ref, b_ref, out_ref):
    out_ref[...] = jnp.dot(v_ref[...], w_ref[...],
                           preferred_element_type=jnp.float32) + b_ref[...]


def _linear(v, wp, bp, bn):
    kdim = wp.shape[0]
    ndim = wp.shape[1]
    return pl.pallas_call(
        _lin_body,
        grid=(ndim // bn,),
        in_specs=[
            pl.BlockSpec((1, kdim), lambda j: (0, 0)),
            pl.BlockSpec((kdim, bn), lambda j: (0, j)),
            pl.BlockSpec((1, bn), lambda j: (0, j)),
        ],
        out_specs=pl.BlockSpec((1, bn), lambda j: (0, j)),
        out_shape=jax.ShapeDtypeStruct((1, ndim), jnp.float32),
    )(v, wp, bp)


def kernel(x, edge_index, batch, W1, b1, W2, b2, W3, b3, Wl, bl):
    f32 = jnp.float32
    src = edge_index[0]
    dst = edge_index[1]

    deg = jnp.zeros((N,), f32).at[dst].add(1.0) + 1.0
    dis = lax.rsqrt(deg)
    u = dis * (jnp.zeros((N,), f32).at[src].add(dis[dst]) + dis) * (1.0 / N)

    disp = jnp.pad(dis, (0, NP - N))
    up = jnp.pad(u, (0, NP - N))
    disc = disp[:, None]
    uc = up[:, None]

    xp = jnp.pad(x, ((0, NP - N), (0, 0)))
    w1b = jnp.pad(W1, ((0, 0), (0, H1P - H1))).astype(jnp.bfloat16)
    b1p = jnp.pad(b1, (0, H1P - H1))[None, :]
    w2b = jnp.pad(W2, ((0, H1P - H1), (0, H2P - H2))).astype(jnp.bfloat16)
    b2r = jnp.pad(b2, (0, H2P - H2)).reshape(NCHUNK, 1, F)
    w3p = jnp.pad(W3, ((0, H2P - H2), (0, 2048 - 2000)))
    b3p = jnp.pad(b3, (0, 2048 - 2000))[None, :]
    wlp = jnp.pad(Wl, ((0, 2048 - 2000), (0, 128 - 10)))
    blp = jnp.pad(bl, (0, 128 - 10))[None, :]

    srcp = jnp.pad(src, (0, EP - E), constant_values=NP - 1)
    dstp = jnp.pad(dst, (0, EP - E), constant_values=NP - 1)
    src1 = srcp.reshape(NC * NS, E1_NB, E1_BATCH)
    dst1 = dstp.reshape(NC * NS, E1_NB, E1_BATCH)
    src2 = srcp.reshape(NS, E2_NB, E2_BATCH)
    dst2 = dstp.reshape(NS, E2_NB, E2_BATCH)
    zeros_z = jnp.zeros((ROWS_PER_TILE, F), f32)

    xs = _scale_rows(xp, disc)
    parts = _sc1(xs, src1, dst1, zeros_z)
    h1r = _mm1(parts, xs, disc, w1b, b1p)

    ms = _mm2(h1r, w2b, disc)
    src2c = src2[None] + (jnp.arange(NCHUNK, dtype=jnp.int32) * NP)[:, None, None, None]
    qh = _sc2(ms.reshape(NCHUNK * NP, F), src2c, dst2, zeros_z)

    v = _vred(qh, ms, uc, disc, b2r).reshape(1, H2P)
    t = _linear(v, w3p, b3p, 512)
    out = _linear(t, wlp, blp, 128)
    return out[:, :10]


# speedup vs baseline: 13.6553x; 1.0312x over previous
import functools

import jax
import jax.numpy as jnp
from jax import lax
from jax.experimental import pallas as pl
from jax.experimental.pallas import tpu as pltpu
from jax.experimental.pallas import tpu_sc as plsc

N = 10000
NP = 10240
E = 160000
D = 128
H1 = 9000
H1P = 9216
H2 = 6000
H2P = 6144
F = 128
NCHUNK = H2P // F
NC = 2
NS = 16
ROWS_PER_TILE = NP // NS
CH_PER_SC = NCHUNK // NC

EP = 163840
E1_BATCH = 256
E1_NB = EP // (NC * NS * E1_BATCH)
E2_BATCH = 128
E2_NB = EP // (NS * E2_BATCH)
NBUF2 = 2


def _sc1_body(xs_hbm, src_hbm, dst_hbm, zeros_hbm, out_hbm,
              src_v, dst_v, gbuf, acc, sem):
    cid = lax.axis_index("c")
    sid = lax.axis_index("s")
    wid = cid * NS + sid
    base = sid * ROWS_PER_TILE
    pltpu.sync_copy(zeros_hbm, acc.at[pl.ds(base, ROWS_PER_TILE)])
    plsc.subcore_barrier()

    def edge_batch(j, carry):
        pltpu.sync_copy(src_hbm.at[wid, j], src_v)
        pltpu.sync_copy(dst_hbm.at[wid, j], dst_v)
        pltpu.async_copy(xs_hbm.at[src_v], gbuf, sem).wait()
        pltpu.sync_copy(gbuf, acc.at[dst_v], add=True)
        return carry
    lax.fori_loop(0, E1_NB, edge_batch, 0)
    plsc.subcore_barrier()
    pltpu.sync_copy(acc.at[pl.ds(base, ROWS_PER_TILE)],
                    out_hbm.at[cid].at[pl.ds(base, ROWS_PER_TILE)])


@functools.cache
def _make_sc1():
    return pl.kernel(
        _sc1_body,
        out_type=jax.ShapeDtypeStruct((NC, NP, D), jnp.float32),
        mesh=plsc.VectorSubcoreMesh(core_axis_name="c", subcore_axis_name="s",
                                    num_cores=NC, num_subcores=NS),
        scratch_types=[
            pltpu.VMEM((E1_BATCH,), jnp.int32),
            pltpu.VMEM((E1_BATCH,), jnp.int32),
            pltpu.VMEM((E1_BATCH, D), jnp.float32),
            pltpu.VMEM_SHARED((NP, D), jnp.float32),
            pltpu.SemaphoreType.DMA,
        ],
    )


def _sc1(*args):
    return _make_sc1()(*args)


def _sc2_body(ms_hbm, src_hbm, dst_hbm, zeros_hbm, out_hbm,
              srcv0, srcv1, gbuf0, gbuf1, dstall, acc,
              isem0, isem1, gsem0, gsem1, ssem0, ssem1):
    cid = lax.axis_index("c")
    sid = lax.axis_index("s")
    base = sid * ROWS_PER_TILE
    srcv = (srcv0, srcv1)
    gbuf = (gbuf0, gbuf1)
    isem = (isem0, isem1)
    gsem = (gsem0, gsem1)
    ssem = (ssem0, ssem1)

    pltpu.sync_copy(dst_hbm.at[sid], dstall)

    def idx_d(c, j, b):
        return pltpu.make_async_copy(src_hbm.at[c, sid, j], srcv[b], isem[b])

    def gather_d(b):
        return pltpu.make_async_copy(ms_hbm.at[srcv[b]], gbuf[b], gsem[b])

    def scatter_d(j, b):
        return pltpu.make_async_copy(gbuf[b], acc.at[dstall.at[j]], ssem[b])

    ngrp = E2_NB // NBUF2

    def per_chunk(t, carry):
        c = cid * CH_PER_SC + t
        pltpu.sync_copy(zeros_hbm, acc.at[pl.ds(base, ROWS_PER_TILE)])
        plsc.subcore_barrier()
        for b in range(NBUF2):
            idx_d(c, b, b).start()

        def group(g, carry2):
            j0 = g * NBUF2
            for b in range(NBUF2):
                idx_d(c, j0 + b, b).wait()

                @pl.when(g > 0)
                def _():
                    scatter_d(j0 + b - NBUF2, b).wait()
                gather_d(b).start()
            for b in range(NBUF2):
                gather_d(b).wait()
                scatter_d(j0 + b, b).start(add=True)
            for b in range(NBUF2):
                jn = jnp.minimum(j0 + NBUF2 + b, E2_NB - 1)
                idx_d(c, jn, b).start()
            return carry2
        lax.fori_loop(0, ngrp, group, 0)
        for b in range(NBUF2):
            idx_d(c, E2_NB - 1, b).wait()
            scatter_d(E2_NB - NBUF2 + b, b).wait()
        plsc.subcore_barrier()
        pltpu.sync_copy(acc.at[pl.ds(base, ROWS_PER_TILE)],
                        out_hbm.at[c].at[pl.ds(base, ROWS_PER_TILE)])
        plsc.subcore_barrier()
        return carry
    lax.fori_loop(0, CH_PER_SC, per_chunk, 0)


@functools.cache
def _make_sc2():
    return pl.kernel(
        _sc2_body,
        out_type=jax.ShapeDtypeStruct((NCHUNK, NP, F), jnp.float32),
        mesh=plsc.VectorSubcoreMesh(core_axis_name="c", subcore_axis_name="s",
                                    num_cores=NC, num_subcores=NS),
        scratch_types=[
            pltpu.VMEM((E2_BATCH,), jnp.int32),
            pltpu.VMEM((E2_BATCH,), jnp.int32),
            pltpu.VMEM((E2_BATCH, F), jnp.float32),
            pltpu.VMEM((E2_BATCH, F), jnp.float32),
            pltpu.VMEM((E2_NB, E2_BATCH), jnp.int32),
            pltpu.VMEM_SHARED((NP, F), jnp.float32),
            pltpu.SemaphoreType.DMA,
            pltpu.SemaphoreType.DMA,
            pltpu.SemaphoreType.DMA,
            pltpu.SemaphoreType.DMA,
            pltpu.SemaphoreType.DMA,
            pltpu.SemaphoreType.DMA,
        ],
    )


def _sc2(*args):
    return _make_sc2()(*args)


def _scale_body(x_ref, dis_ref, out_ref):
    out_ref[...] = x_ref[...] * dis_ref[...]


def _scale_rows(xp, disc):
    bm = 1024
    return pl.pallas_call(
        _scale_body,
        grid=(NP // bm,),
        in_specs=[
            pl.BlockSpec((bm, D), lambda i: (i, 0)),
            pl.BlockSpec((bm, 1), lambda i: (i, 0)),
        ],
        out_specs=pl.BlockSpec((bm, D), lambda i: (i, 0)),
        out_shape=jax.ShapeDtypeStruct((NP, D), jnp.float32),
    )(xp, disc)


def _mm1_body(p_ref, xs_ref, dis_ref, w1_ref, b1_ref, out_ref):
    a = ((p_ref[0] + p_ref[1] + xs_ref[...]) * dis_ref[...]).astype(jnp.bfloat16)
    r = jnp.dot(a, w1_ref[...], preferred_element_type=jnp.float32)
    out_ref[...] = jnp.maximum(r + b1_ref[...], 0.0).astype(jnp.bfloat16)


def _mm1(parts, xs, disc, w1b, b1p):
    bm, bn = 1024, 1024
    return pl.pallas_call(
        _mm1_body,
        grid=(NP // bm, H1P // bn),
        in_specs=[
            pl.BlockSpec((2, bm, D), lambda i, j: (0, i, 0)),
            pl.BlockSpec((bm, D), lambda i, j: (i, 0)),
            pl.BlockSpec((bm, 1), lambda i, j: (i, 0)),
            pl.BlockSpec((D, bn), lambda i, j: (0, j)),
            pl.BlockSpec((1, bn), lambda i, j: (0, j)),
        ],
        out_specs=pl.BlockSpec((bm, bn), lambda i, j: (i, j)),
        out_shape=jax.ShapeDtypeStruct((NP, H1P), jnp.bfloat16),
    )(parts, xs, disc, w1b, b1p)


def _mm2_body(h_ref, w_ref, dis_ref, out_ref, acc_ref, *, nk, nchunk_blk):
    k = pl.program_id(2)

    @pl.when(k == 0)
    def _():
        acc_ref[...] = jnp.zeros_like(acc_ref)

    acc_ref[...] += jnp.dot(h_ref[...], w_ref[...],
                            preferred_element_type=jnp.float32)

    @pl.when(k == nk - 1)
    def _():
        s = acc_ref[...] * dis_ref[...]
        for c in range(nchunk_blk):
            out_ref[c] = s[:, c * F:(c + 1) * F]


def _mm2(h1r, w2b, disc):
    bm, bn, bk = 1024, H2P // 2, 512
    nk = H1P // bk
    nchunk_blk = bn // F
    return pl.pallas_call(
        functools.partial(_mm2_body, nk=nk, nchunk_blk=nchunk_blk),
        grid=(NP // bm, H2P // bn, nk),
        in_specs=[
            pl.BlockSpec((bm, bk), lambda i, j, k: (i, k)),
            pl.BlockSpec((bk, bn), lambda i, j, k: (k, j)),
            pl.BlockSpec((bm, 1), lambda i, j, k: (i, 0)),
        ],
        out_specs=pl.BlockSpec((nchunk_blk, bm, F), lambda i, j, k: (j, i, 0)),
        out_shape=jax.ShapeDtypeStruct((NCHUNK, NP, F), jnp.float32),
        scratch_shapes=[pltpu.VMEM((bm, bn), jnp.float32)],
    )(h1r, w2b, disc)


def _vred_body(qh_ref, ms_ref, u_ref, dis_ref, b2_ref, out_ref):
    i = pl.program_id(1)
    h = jnp.maximum(dis_ref[...] * (qh_ref[0] + ms_ref[0]) + b2_ref[0], 0.0)
    contrib = jnp.sum(u_ref[...] * h, axis=0, keepdims=True)[None]

    @pl.when(i == 0)
    def _():
        out_ref[...] = contrib

    @pl.when(i > 0)
    def _():
        out_ref[...] += contrib


def _vred(qh, ms, uc, disc, b2r):
    bm = 2048
    return pl.pallas_call(
        _vred_body,
        grid=(NCHUNK, NP // bm),
        in_specs=[
            pl.BlockSpec((1, bm, F), lambda c, i: (c, i, 0)),
            pl.BlockSpec((1, bm, F), lambda c, i: (c, i, 0)),
            pl.BlockSpec((bm, 1), lambda c, i: (i, 0)),
            pl.BlockSpec((bm, 1), lambda c, i: (i, 0)),
            pl.BlockSpec((1, 1, F), lambda c, i: (c, 0, 0)),
        ],
        out_specs=pl.BlockSpec((1, 1, F), lambda c, i: (c, 0, 0)),
        out_shape=jax.ShapeDtypeStruct((NCHUNK, 1, F), jnp.float32),
    )(qh, ms, uc, disc, b2r)


def _lin_body(v_ref, w_ref, b_ref, out_ref):
    out_ref[...] = jnp.dot(v_ref[...], w_ref[...],
                           preferred_element_type=jnp.float32) + b_ref[...]


def _linear(v, wp, bp, bn):
    kdim = wp.shape[0]
    ndim = wp.shape[1]
    return pl.pallas_call(
        _lin_body,
        grid=(ndim // bn,),
        in_specs=[
            pl.BlockSpec((1, kdim), lambda j: (0, 0)),
            pl.BlockSpec((kdim, bn), lambda j: (0, j)),
            pl.BlockSpec((1, bn), lambda j: (0, j)),
        ],
        out_specs=pl.BlockSpec((1, bn), lambda j: (0, j)),
        out_shape=jax.ShapeDtypeStruct((1, ndim), jnp.float32),
    )(v, wp, bp)


def kernel(x, edge_index, batch, W1, b1, W2, b2, W3, b3, Wl, bl):
    f32 = jnp.float32
    src = edge_index[0]
    dst = edge_index[1]

    deg = jnp.zeros((N,), f32).at[dst].add(1.0) + 1.0
    dis = lax.rsqrt(deg)
    u = dis * (jnp.zeros((N,), f32).at[src].add(dis[dst]) + dis) * (1.0 / N)

    disp = jnp.pad(dis, (0, NP - N))
    up = jnp.pad(u, (0, NP - N))
    disc = disp[:, None]
    uc = up[:, None]

    xp = jnp.pad(x, ((0, NP - N), (0, 0)))
    w1b = jnp.pad(W1, ((0, 0), (0, H1P - H1))).astype(jnp.bfloat16)
    b1p = jnp.pad(b1, (0, H1P - H1))[None, :]
    w2b = jnp.pad(W2, ((0, H1P - H1), (0, H2P - H2))).astype(jnp.bfloat16)
    b2r = jnp.pad(b2, (0, H2P - H2)).reshape(NCHUNK, 1, F)
    w3p = jnp.pad(W3, ((0, H2P - H2), (0, 2048 - 2000)))
    b3p = jnp.pad(b3, (0, 2048 - 2000))[None, :]
    wlp = jnp.pad(Wl, ((0, 2048 - 2000), (0, 128 - 10)))
    blp = jnp.pad(bl, (0, 128 - 10))[None, :]

    srcp = jnp.pad(src, (0, EP - E), constant_values=NP - 1)
    dstp = jnp.pad(dst, (0, EP - E), constant_values=NP - 1)
    src1 = srcp.reshape(NC * NS, E1_NB, E1_BATCH)
    dst1 = dstp.reshape(NC * NS, E1_NB, E1_BATCH)
    src2 = srcp.reshape(NS, E2_NB, E2_BATCH)
    dst2 = dstp.reshape(NS, E2_NB, E2_BATCH)
    zeros_z = jnp.zeros((ROWS_PER_TILE, F), f32)

    xs = _scale_rows(xp, disc)
    parts = _sc1(xs, src1, dst1, zeros_z)
    h1r = _mm1(parts, xs, disc, w1b, b1p)

    ms = _mm2(h1r, w2b, disc)
    src2c = src2[None] + (jnp.arange(NCHUNK, dtype=jnp.int32) * NP)[:, None, None, None]
    qh = _sc2(ms.reshape(NCHUNK * NP, F), src2c, dst2, zeros_z)

    v = _vred(qh, ms, uc, disc, b2r).reshape(1, H2P)
    t = _linear(v, w3p, b3p, 512)
    out = _linear(t, wlp, blp, 128)
    return out[:, :10]


# speedup vs baseline: 14.1309x; 1.0348x over previous
import functools

import jax
import jax.numpy as jnp
from jax import lax
from jax.experimental import pallas as pl
from jax.experimental.pallas import tpu as pltpu
from jax.experimental.pallas import tpu_sc as plsc

N = 10000
NP = 10240
E = 160000
D = 128
H1 = 9000
H1P = 9216
H2 = 6000
H2P = 6144
F = 128
NCHUNK = H2P // F
NC = 2
NS = 16
ROWS_PER_TILE = NP // NS
CH_PER_SC = NCHUNK // NC

EP = 163840
E1_BATCH = 256
E1_NB = EP // (NC * NS * E1_BATCH)
E2_BATCH = 128
E2_NB = EP // (NS * E2_BATCH)
NBUF2 = 2
NCHUNK_H = NCHUNK // 2
CH_PER_SC_H = NCHUNK_H // NC


def _sc1_body(xs_hbm, src_hbm, dst_hbm, zeros_hbm, out_hbm,
              src_v, dst_v, gbuf, acc, sem):
    cid = lax.axis_index("c")
    sid = lax.axis_index("s")
    wid = cid * NS + sid
    base = sid * ROWS_PER_TILE
    pltpu.sync_copy(zeros_hbm, acc.at[pl.ds(base, ROWS_PER_TILE)])
    plsc.subcore_barrier()

    def edge_batch(j, carry):
        pltpu.sync_copy(src_hbm.at[wid, j], src_v)
        pltpu.sync_copy(dst_hbm.at[wid, j], dst_v)
        pltpu.async_copy(xs_hbm.at[src_v], gbuf, sem).wait()
        pltpu.sync_copy(gbuf, acc.at[dst_v], add=True)
        return carry
    lax.fori_loop(0, E1_NB, edge_batch, 0)
    plsc.subcore_barrier()
    pltpu.sync_copy(acc.at[pl.ds(base, ROWS_PER_TILE)],
                    out_hbm.at[cid].at[pl.ds(base, ROWS_PER_TILE)])


@functools.cache
def _make_sc1():
    return pl.kernel(
        _sc1_body,
        out_type=jax.ShapeDtypeStruct((NC, NP, D), jnp.float32),
        mesh=plsc.VectorSubcoreMesh(core_axis_name="c", subcore_axis_name="s",
                                    num_cores=NC, num_subcores=NS),
        scratch_types=[
            pltpu.VMEM((E1_BATCH,), jnp.int32),
            pltpu.VMEM((E1_BATCH,), jnp.int32),
            pltpu.VMEM((E1_BATCH, D), jnp.float32),
            pltpu.VMEM_SHARED((NP, D), jnp.float32),
            pltpu.SemaphoreType.DMA,
        ],
    )


def _sc1(*args):
    return _make_sc1()(*args)


def _sc2_body(ms_hbm, src_hbm, dst_hbm, zeros_hbm, out_hbm,
              srcv0, srcv1, gbuf0, gbuf1, dstall, acc,
              isem0, isem1, gsem0, gsem1, ssem0, ssem1):
    cid = lax.axis_index("c")
    sid = lax.axis_index("s")
    base = sid * ROWS_PER_TILE
    srcv = (srcv0, srcv1)
    gbuf = (gbuf0, gbuf1)
    isem = (isem0, isem1)
    gsem = (gsem0, gsem1)
    ssem = (ssem0, ssem1)

    pltpu.sync_copy(dst_hbm.at[sid], dstall)

    def idx_d(j, b):
        return pltpu.make_async_copy(src_hbm.at[sid, j], srcv[b], isem[b])

    def gather_d(b):
        return pltpu.make_async_copy(ms_hbm.at[srcv[b]], gbuf[b], gsem[b])

    def scatter_d(j, b):
        return pltpu.make_async_copy(gbuf[b], acc.at[dstall.at[j]], ssem[b])

    ngrp = E2_NB // NBUF2

    def per_chunk(t, carry):
        c = cid * CH_PER_SC_H + t
        shift = c * NP
        pltpu.sync_copy(zeros_hbm, acc.at[pl.ds(base, ROWS_PER_TILE)])
        plsc.subcore_barrier()
        for b in range(NBUF2):
            idx_d(b, b).start()

        def group(g, carry2):
            j0 = g * NBUF2
            for b in range(NBUF2):
                idx_d(j0 + b, b).wait()
                for k in range(E2_BATCH // 16):
                    sl = pl.ds(k * 16, 16)
                    srcv[b][sl] = srcv[b][sl] + shift

                @pl.when(g > 0)
                def _():
                    scatter_d(j0 + b - NBUF2, b).wait()
                gather_d(b).start()
            for b in range(NBUF2):
                gather_d(b).wait()
                scatter_d(j0 + b, b).start(add=True)
            for b in range(NBUF2):
                jn = jnp.minimum(j0 + NBUF2 + b, E2_NB - 1)
                idx_d(jn, b).start()
            return carry2
        lax.fori_loop(0, ngrp, group, 0)
        for b in range(NBUF2):
            idx_d(E2_NB - 1, b).wait()
            scatter_d(E2_NB - NBUF2 + b, b).wait()
        plsc.subcore_barrier()
        pltpu.sync_copy(acc.at[pl.ds(base, ROWS_PER_TILE)],
                        out_hbm.at[c].at[pl.ds(base, ROWS_PER_TILE)])
        plsc.subcore_barrier()
        return carry
    lax.fori_loop(0, CH_PER_SC_H, per_chunk, 0)


@functools.cache
def _make_sc2():
    return pl.kernel(
        _sc2_body,
        out_type=jax.ShapeDtypeStruct((NCHUNK_H, NP, F), jnp.float32),
        mesh=plsc.VectorSubcoreMesh(core_axis_name="c", subcore_axis_name="s",
                                    num_cores=NC, num_subcores=NS),
        scratch_types=[
            pltpu.VMEM((E2_BATCH,), jnp.int32),
            pltpu.VMEM((E2_BATCH,), jnp.int32),
            pltpu.VMEM((E2_BATCH, F), jnp.float32),
            pltpu.VMEM((E2_BATCH, F), jnp.float32),
            pltpu.VMEM((E2_NB, E2_BATCH), jnp.int32),
            pltpu.VMEM_SHARED((NP, F), jnp.float32),
            pltpu.SemaphoreType.DMA,
            pltpu.SemaphoreType.DMA,
            pltpu.SemaphoreType.DMA,
            pltpu.SemaphoreType.DMA,
            pltpu.SemaphoreType.DMA,
            pltpu.SemaphoreType.DMA,
        ],
    )


def _sc2(*args):
    return _make_sc2()(*args)


def _scale_body(x_ref, dis_ref, out_ref):
    out_ref[...] = x_ref[...] * dis_ref[...]


def _scale_rows(xp, disc):
    bm = 1024
    return pl.pallas_call(
        _scale_body,
        grid=(NP // bm,),
        in_specs=[
            pl.BlockSpec((bm, D), lambda i: (i, 0)),
            pl.BlockSpec((bm, 1), lambda i: (i, 0)),
        ],
        out_specs=pl.BlockSpec((bm, D), lambda i: (i, 0)),
        out_shape=jax.ShapeDtypeStruct((NP, D), jnp.float32),
    )(xp, disc)


def _mm1_body(p_ref, xs_ref, dis_ref, w1_ref, b1_ref, out_ref):
    a = ((p_ref[0] + p_ref[1] + xs_ref[...]) * dis_ref[...]).astype(jnp.bfloat16)
    r = jnp.dot(a, w1_ref[...], preferred_element_type=jnp.float32)
    out_ref[...] = jnp.maximum(r + b1_ref[...], 0.0).astype(jnp.bfloat16)


def _mm1(parts, xs, disc, w1b, b1p):
    bm, bn = 1024, 1024
    return pl.pallas_call(
        _mm1_body,
        grid=(NP // bm, H1P // bn),
        in_specs=[
            pl.BlockSpec((2, bm, D), lambda i, j: (0, i, 0)),
            pl.BlockSpec((bm, D), lambda i, j: (i, 0)),
            pl.BlockSpec((bm, 1), lambda i, j: (i, 0)),
            pl.BlockSpec((D, bn), lambda i, j: (0, j)),
            pl.BlockSpec((1, bn), lambda i, j: (0, j)),
        ],
        out_specs=pl.BlockSpec((bm, bn), lambda i, j: (i, j)),
        out_shape=jax.ShapeDtypeStruct((NP, H1P), jnp.bfloat16),
    )(parts, xs, disc, w1b, b1p)


def _mm2_body(h_ref, w_ref, dis_ref, out_ref, acc_ref, *, nk, nchunk_blk):
    k = pl.program_id(2)

    @pl.when(k == 0)
    def _():
        acc_ref[...] = jnp.zeros_like(acc_ref)

    acc_ref[...] += jnp.dot(h_ref[...], w_ref[...],
                            preferred_element_type=jnp.float32)

    @pl.when(k == nk - 1)
    def _():
        s = acc_ref[...] * dis_ref[...]
        for c in range(nchunk_blk):
            out_ref[c] = s[:, c * F:(c + 1) * F]


def _mm2(h1r, w2b, disc, jhalf):
    bm, bn, bk = 1024, H2P // 2, 512
    nk = H1P // bk
    nchunk_blk = bn // F
    return pl.pallas_call(
        functools.partial(_mm2_body, nk=nk, nchunk_blk=nchunk_blk),
        grid=(NP // bm, 1, nk),
        in_specs=[
            pl.BlockSpec((bm, bk), lambda i, j, k: (i, k)),
            pl.BlockSpec((bk, bn), lambda i, j, k, _j=jhalf: (k, _j)),
            pl.BlockSpec((bm, 1), lambda i, j, k: (i, 0)),
        ],
        out_specs=pl.BlockSpec((nchunk_blk, bm, F), lambda i, j, k: (0, i, 0)),
        out_shape=jax.ShapeDtypeStruct((NCHUNK_H, NP, F), jnp.float32),
        scratch_shapes=[pltpu.VMEM((bm, bn), jnp.float32)],
    )(h1r, w2b, disc)


def _vred_body(qh_ref, ms_ref, u_ref, dis_ref, b2_ref, out_ref):
    i = pl.program_id(1)
    h = jnp.maximum(dis_ref[...] * (qh_ref[0] + ms_ref[0]) + b2_ref[0], 0.0)
    contrib = jnp.sum(u_ref[...] * h, axis=0, keepdims=True)[None]

    @pl.when(i == 0)
    def _():
        out_ref[...] = contrib

    @pl.when(i > 0)
    def _():
        out_ref[...] += contrib


def _vred(qh, ms, uc, disc, b2r):
    bm = 2048
    return pl.pallas_call(
        _vred_body,
        grid=(NCHUNK_H, NP // bm),
        in_specs=[
            pl.BlockSpec((1, bm, F), lambda c, i: (c, i, 0)),
            pl.BlockSpec((1, bm, F), lambda c, i: (c, i, 0)),
            pl.BlockSpec((bm, 1), lambda c, i: (i, 0)),
            pl.BlockSpec((bm, 1), lambda c, i: (i, 0)),
            pl.BlockSpec((1, 1, F), lambda c, i: (c, 0, 0)),
        ],
        out_specs=pl.BlockSpec((1, 1, F), lambda c, i: (c, 0, 0)),
        out_shape=jax.ShapeDtypeStruct((NCHUNK_H, 1, F), jnp.float32),
    )(qh, ms, uc, disc, b2r)


def _lin_body(v_ref, w_ref, b_ref, out_ref):
    out_ref[...] = jnp.dot(v_ref[...], w_ref[...],
                           preferred_element_type=jnp.float32) + b_ref[...]


def _linear(v, wp, bp, bn):
    kdim = wp.shape[0]
    ndim = wp.shape[1]
    return pl.pallas_call(
        _lin_body,
        grid=(ndim // bn,),
        in_specs=[
            pl.BlockSpec((1, kdim), lambda j: (0, 0)),
            pl.BlockSpec((kdim, bn), lambda j: (0, j)),
            pl.BlockSpec((1, bn), lambda j: (0, j)),
        ],
        out_specs=pl.BlockSpec((1, bn), lambda j: (0, j)),
        out_shape=jax.ShapeDtypeStruct((1, ndim), jnp.float32),
    )(v, wp, bp)


def kernel(x, edge_index, batch, W1, b1, W2, b2, W3, b3, Wl, bl):
    f32 = jnp.float32
    src = edge_index[0]
    dst = edge_index[1]

    deg = jnp.zeros((N,), f32).at[dst].add(1.0) + 1.0
    dis = lax.rsqrt(deg)
    u = dis * (jnp.zeros((N,), f32).at[src].add(dis[dst]) + dis) * (1.0 / N)

    disp = jnp.pad(dis, (0, NP - N))
    up = jnp.pad(u, (0, NP - N))
    disc = disp[:, None]
    uc = up[:, None]

    xp = jnp.pad(x, ((0, NP - N), (0, 0)))
    w1b = jnp.pad(W1, ((0, 0), (0, H1P - H1))).astype(jnp.bfloat16)
    b1p = jnp.pad(b1, (0, H1P - H1))[None, :]
    w2b = jnp.pad(W2, ((0, H1P - H1), (0, H2P - H2))).astype(jnp.bfloat16)
    b2r = jnp.pad(b2, (0, H2P - H2)).reshape(NCHUNK, 1, F)
    w3p = jnp.pad(W3, ((0, H2P - H2), (0, 2048 - 2000)))
    b3p = jnp.pad(b3, (0, 2048 - 2000))[None, :]
    wlp = jnp.pad(Wl, ((0, 2048 - 2000), (0, 128 - 10)))
    blp = jnp.pad(bl, (0, 128 - 10))[None, :]

    srcp = jnp.pad(src, (0, EP - E), constant_values=NP - 1)
    dstp = jnp.pad(dst, (0, EP - E), constant_values=NP - 1)
    src1 = srcp.reshape(NC * NS, E1_NB, E1_BATCH)
    dst1 = dstp.reshape(NC * NS, E1_NB, E1_BATCH)
    src2 = srcp.reshape(NS, E2_NB, E2_BATCH)
    dst2 = dstp.reshape(NS, E2_NB, E2_BATCH)
    zeros_z = jnp.zeros((ROWS_PER_TILE, F), f32)

    xs = _scale_rows(xp, disc)
    parts = _sc1(xs, src1, dst1, zeros_z)
    h1r = _mm1(parts, xs, disc, w1b, b1p)

    ms_a = _mm2(h1r, w2b, disc, 0)
    qh_a = _sc2(ms_a.reshape(NCHUNK_H * NP, F), src2, dst2, zeros_z)
    ms_b = _mm2(h1r, w2b, disc, 1)
    qh_b = _sc2(ms_b.reshape(NCHUNK_H * NP, F), src2, dst2, zeros_z)

    v_a = _vred(qh_a, ms_a, uc, disc, b2r[:NCHUNK_H])
    v_b = _vred(qh_b, ms_b, uc, disc, b2r[NCHUNK_H:])
    v = jnp.concatenate([v_a, v_b]).reshape(1, H2P)
    t = _linear(v, w3p, b3p, 512)
    out = _linear(t, wlp, blp, 128)
    return out[:, :10]


# speedup vs baseline: 21.8702x; 1.5477x over previous
import functools

import jax
import jax.numpy as jnp
from jax import lax
from jax.experimental import pallas as pl
from jax.experimental.pallas import tpu as pltpu
from jax.experimental.pallas import tpu_sc as plsc

N = 10000
NP = 10240
E = 160000
D = 128
H1 = 9000
H1P = 9216
H2 = 6000
H2P = 6144
F = 128
NCHUNK = H2P // F
NC = 2
NS = 16
ROWS_PER_TILE = NP // NS
CH_PER_SC = NCHUNK // NC

EP = 163840
E1_BATCH = 256
E1_NB = EP // (NC * NS * E1_BATCH)
E2_BATCH = 128
E2_NB = EP // (NS * E2_BATCH)
NBUF2 = 2
NCHUNK_H = NCHUNK // 2
CH_PER_SC_H = NCHUNK_H // NC


def _sc1_body(xs_hbm, src_hbm, dst_hbm, zeros_hbm, out_hbm,
              src_v, dst_v, gbuf, acc, sem):
    cid = lax.axis_index("c")
    sid = lax.axis_index("s")
    wid = cid * NS + sid
    base = sid * ROWS_PER_TILE
    pltpu.sync_copy(zeros_hbm, acc.at[pl.ds(base, ROWS_PER_TILE)])
    plsc.subcore_barrier()

    def edge_batch(j, carry):
        pltpu.sync_copy(src_hbm.at[wid, j], src_v)
        pltpu.sync_copy(dst_hbm.at[wid, j], dst_v)
        pltpu.async_copy(xs_hbm.at[src_v], gbuf, sem).wait()
        pltpu.sync_copy(gbuf, acc.at[dst_v], add=True)
        return carry
    lax.fori_loop(0, E1_NB, edge_batch, 0)
    plsc.subcore_barrier()
    pltpu.sync_copy(acc.at[pl.ds(base, ROWS_PER_TILE)],
                    out_hbm.at[cid].at[pl.ds(base, ROWS_PER_TILE)])


@functools.cache
def _make_sc1():
    return pl.kernel(
        _sc1_body,
        out_type=jax.ShapeDtypeStruct((NC, NP, D), jnp.float32),
        mesh=plsc.VectorSubcoreMesh(core_axis_name="c", subcore_axis_name="s",
                                    num_cores=NC, num_subcores=NS),
        scratch_types=[
            pltpu.VMEM((E1_BATCH,), jnp.int32),
            pltpu.VMEM((E1_BATCH,), jnp.int32),
            pltpu.VMEM((E1_BATCH, D), jnp.float32),
            pltpu.VMEM_SHARED((NP, D), jnp.float32),
            pltpu.SemaphoreType.DMA,
        ],
    )


def _sc1(*args):
    return _make_sc1()(*args)


def _sc2_body(ms_hbm, src_hbm, dst_hbm, zeros_hbm, out_hbm,
              srcv0, srcv1, gbuf0, gbuf1, dstall, acc,
              isem0, isem1, gsem0, gsem1, ssem0, ssem1):
    cid = lax.axis_index("c")
    sid = lax.axis_index("s")
    base = sid * ROWS_PER_TILE
    srcv = (srcv0, srcv1)
    gbuf = (gbuf0, gbuf1)
    isem = (isem0, isem1)
    gsem = (gsem0, gsem1)
    ssem = (ssem0, ssem1)

    pltpu.sync_copy(dst_hbm.at[sid], dstall)

    def idx_d(j, b):
        return pltpu.make_async_copy(src_hbm.at[sid, j], srcv[b], isem[b])

    def gather_d(b):
        return pltpu.make_async_copy(ms_hbm.at[srcv[b]], gbuf[b], gsem[b])

    def scatter_d(j, b):
        return pltpu.make_async_copy(gbuf[b], acc.at[dstall.at[j]], ssem[b])

    ngrp = E2_NB // NBUF2

    def per_chunk(t, carry):
        c = cid * CH_PER_SC_H + t
        shift = c * NP
        pltpu.sync_copy(zeros_hbm, acc.at[pl.ds(base, ROWS_PER_TILE)])
        plsc.subcore_barrier()
        for b in range(NBUF2):
            idx_d(b, b).start()

        def group(g, carry2):
            j0 = g * NBUF2
            for b in range(NBUF2):
                idx_d(j0 + b, b).wait()
                for k in range(E2_BATCH // 16):
                    sl = pl.ds(k * 16, 16)
                    srcv[b][sl] = srcv[b][sl] + shift

                @pl.when(g > 0)
                def _():
                    scatter_d(j0 + b - NBUF2, b).wait()
                gather_d(b).start()
            for b in range(NBUF2):
                gather_d(b).wait()
                scatter_d(j0 + b, b).start(add=True)
            for b in range(NBUF2):
                jn = jnp.minimum(j0 + NBUF2 + b, E2_NB - 1)
                idx_d(jn, b).start()
            return carry2
        lax.fori_loop(0, ngrp, group, 0)
        for b in range(NBUF2):
            idx_d(E2_NB - 1, b).wait()
            scatter_d(E2_NB - NBUF2 + b, b).wait()
        plsc.subcore_barrier()
        pltpu.sync_copy(acc.at[pl.ds(base, ROWS_PER_TILE)],
                        out_hbm.at[c].at[pl.ds(base, ROWS_PER_TILE)])
        plsc.subcore_barrier()
        return carry
    lax.fori_loop(0, CH_PER_SC_H, per_chunk, 0)


@functools.cache
def _make_sc2():
    return pl.kernel(
        _sc2_body,
        out_type=jax.ShapeDtypeStruct((NCHUNK_H, NP, F), jnp.float32),
        mesh=plsc.VectorSubcoreMesh(core_axis_name="c", subcore_axis_name="s",
                                    num_cores=NC, num_subcores=NS),
        scratch_types=[
            pltpu.VMEM((E2_BATCH,), jnp.int32),
            pltpu.VMEM((E2_BATCH,), jnp.int32),
            pltpu.VMEM((E2_BATCH, F), jnp.float32),
            pltpu.VMEM((E2_BATCH, F), jnp.float32),
            pltpu.VMEM((E2_NB, E2_BATCH), jnp.int32),
            pltpu.VMEM_SHARED((NP, F), jnp.float32),
            pltpu.SemaphoreType.DMA,
            pltpu.SemaphoreType.DMA,
            pltpu.SemaphoreType.DMA,
            pltpu.SemaphoreType.DMA,
            pltpu.SemaphoreType.DMA,
            pltpu.SemaphoreType.DMA,
        ],
    )


def _sc2(*args):
    return _make_sc2()(*args)


def _scale_body(x_ref, dis_ref, out_ref):
    out_ref[...] = x_ref[...] * dis_ref[...]


def _scale_rows(xp, disc):
    bm = 1024
    return pl.pallas_call(
        _scale_body,
        grid=(NP // bm,),
        in_specs=[
            pl.BlockSpec((bm, D), lambda i: (i, 0)),
            pl.BlockSpec((bm, 1), lambda i: (i, 0)),
        ],
        out_specs=pl.BlockSpec((bm, D), lambda i: (i, 0)),
        out_shape=jax.ShapeDtypeStruct((NP, D), jnp.float32),
    )(xp, disc)


def _mm1_body(p_ref, xs_ref, dis_ref, w1_ref, b1_ref, out_ref):
    a = ((p_ref[0] + p_ref[1] + xs_ref[...]) * dis_ref[...]).astype(jnp.bfloat16)
    r = jnp.dot(a, w1_ref[...], preferred_element_type=jnp.float32)
    out_ref[...] = jnp.maximum(r + b1_ref[...], 0.0).astype(jnp.bfloat16)


def _mm1(parts, xs, disc, w1b, b1p):
    bm, bn = 1024, 1024
    return pl.pallas_call(
        _mm1_body,
        grid=(NP // bm, H1P // bn),
        in_specs=[
            pl.BlockSpec((2, bm, D), lambda i, j: (0, i, 0)),
            pl.BlockSpec((bm, D), lambda i, j: (i, 0)),
            pl.BlockSpec((bm, 1), lambda i, j: (i, 0)),
            pl.BlockSpec((D, bn), lambda i, j: (0, j)),
            pl.BlockSpec((1, bn), lambda i, j: (0, j)),
        ],
        out_specs=pl.BlockSpec((bm, bn), lambda i, j: (i, j)),
        out_shape=jax.ShapeDtypeStruct((NP, H1P), jnp.bfloat16),
    )(parts, xs, disc, w1b, b1p)


def _mm2_body(h_ref, w_ref, dis_ref, out_ref, out2_ref, acc_ref, *,
              nk, nchunk_blk):
    k = pl.program_id(2)

    @pl.when(k == 0)
    def _():
        acc_ref[...] = jnp.zeros_like(acc_ref)

    acc_ref[...] += jnp.dot(h_ref[...], w_ref[...],
                            preferred_element_type=jnp.float32)

    @pl.when(k == nk - 1)
    def _():
        s = acc_ref[...] * dis_ref[...]
        for c in range(nchunk_blk):
            out_ref[c] = s[:, c * F:(c + 1) * F]
        out2_ref[...] = s.astype(jnp.bfloat16)


def _mm2(h1r, w2b, disc, jhalf):
    bm, bn, bk = 1024, H2P // 2, 512
    nk = H1P // bk
    nchunk_blk = bn // F
    return pl.pallas_call(
        functools.partial(_mm2_body, nk=nk, nchunk_blk=nchunk_blk),
        grid=(NP // bm, 1, nk),
        in_specs=[
            pl.BlockSpec((bm, bk), lambda i, j, k: (i, k)),
            pl.BlockSpec((bk, bn), lambda i, j, k, _j=jhalf: (k, _j)),
            pl.BlockSpec((bm, 1), lambda i, j, k: (i, 0)),
        ],
        out_specs=[
            pl.BlockSpec((nchunk_blk, bm, F), lambda i, j, k: (0, i, 0)),
            pl.BlockSpec((bm, bn), lambda i, j, k: (i, 0)),
        ],
        out_shape=[
            jax.ShapeDtypeStruct((NCHUNK_H, NP, F), jnp.float32),
            jax.ShapeDtypeStruct((NP, H2P // 2), jnp.bfloat16),
        ],
        scratch_shapes=[pltpu.VMEM((bm, bn), jnp.float32)],
    )(h1r, w2b, disc)


def _amm_body(a_ref, m_ref, out_ref, acc_ref, *, nk, nchunk_blk):
    k = pl.program_id(1)

    @pl.when(k == 0)
    def _():
        acc_ref[...] = jnp.zeros_like(acc_ref)

    acc_ref[...] += jnp.dot(a_ref[...], m_ref[...],
                            preferred_element_type=jnp.float32)

    @pl.when(k == nk - 1)
    def _():
        s = acc_ref[...]
        for c in range(nchunk_blk):
            out_ref[c] = s[:, c * F:(c + 1) * F]


def _amm(ab, msb):
    bm, bk = 1024, 1024
    bn = H2P // 2
    nk = NP // bk
    nchunk_blk = bn // F
    return pl.pallas_call(
        functools.partial(_amm_body, nk=nk, nchunk_blk=nchunk_blk),
        grid=(NP // bm, nk),
        in_specs=[
            pl.BlockSpec((bm, bk), lambda i, k: (i, k)),
            pl.BlockSpec((bk, bn), lambda i, k: (k, 0)),
        ],
        out_specs=pl.BlockSpec((nchunk_blk, bm, F), lambda i, k: (0, i, 0)),
        out_shape=jax.ShapeDtypeStruct((NCHUNK_H, NP, F), jnp.float32),
        scratch_shapes=[pltpu.VMEM((bm, bn), jnp.float32)],
    )(ab, msb)


def _vred_body(qh_ref, ms_ref, u_ref, dis_ref, b2_ref, out_ref):
    i = pl.program_id(1)
    h = jnp.maximum(dis_ref[...] * (qh_ref[0] + ms_ref[0]) + b2_ref[0], 0.0)
    contrib = jnp.sum(u_ref[...] * h, axis=0, keepdims=True)[None]

    @pl.when(i == 0)
    def _():
        out_ref[...] = contrib

    @pl.when(i > 0)
    def _():
        out_ref[...] += contrib


def _vred(qh, ms, uc, disc, b2r):
    bm = 2048
    return pl.pallas_call(
        _vred_body,
        grid=(NCHUNK_H, NP // bm),
        in_specs=[
            pl.BlockSpec((1, bm, F), lambda c, i: (c, i, 0)),
            pl.BlockSpec((1, bm, F), lambda c, i: (c, i, 0)),
            pl.BlockSpec((bm, 1), lambda c, i: (i, 0)),
            pl.BlockSpec((bm, 1), lambda c, i: (i, 0)),
            pl.BlockSpec((1, 1, F), lambda c, i: (c, 0, 0)),
        ],
        out_specs=pl.BlockSpec((1, 1, F), lambda c, i: (c, 0, 0)),
        out_shape=jax.ShapeDtypeStruct((NCHUNK_H, 1, F), jnp.float32),
    )(qh, ms, uc, disc, b2r)


def _lin_body(v_ref, w_ref, b_ref, out_ref):
    out_ref[...] = jnp.dot(v_ref[...], w_ref[...],
                           preferred_element_type=jnp.float32) + b_ref[...]


def _linear(v, wp, bp, bn):
    kdim = wp.shape[0]
    ndim = wp.shape[1]
    return pl.pallas_call(
        _lin_body,
        grid=(ndim // bn,),
        in_specs=[
            pl.BlockSpec((1, kdim), lambda j: (0, 0)),
            pl.BlockSpec((kdim, bn), lambda j: (0, j)),
            pl.BlockSpec((1, bn), lambda j: (0, j)),
        ],
        out_specs=pl.BlockSpec((1, bn), lambda j: (0, j)),
        out_shape=jax.ShapeDtypeStruct((1, ndim), jnp.float32),
    )(v, wp, bp)


def kernel(x, edge_index, batch, W1, b1, W2, b2, W3, b3, Wl, bl):
    f32 = jnp.float32
    src = edge_index[0]
    dst = edge_index[1]

    deg = jnp.zeros((N,), f32).at[dst].add(1.0) + 1.0
    dis = lax.rsqrt(deg)
    u = dis * (jnp.zeros((N,), f32).at[src].add(dis[dst]) + dis) * (1.0 / N)

    disp = jnp.pad(dis, (0, NP - N))
    up = jnp.pad(u, (0, NP - N))
    disc = disp[:, None]
    uc = up[:, None]

    xp = jnp.pad(x, ((0, NP - N), (0, 0)))
    w1b = jnp.pad(W1, ((0, 0), (0, H1P - H1))).astype(jnp.bfloat16)
    b1p = jnp.pad(b1, (0, H1P - H1))[None, :]
    w2b = jnp.pad(W2, ((0, H1P - H1), (0, H2P - H2))).astype(jnp.bfloat16)
    b2r = jnp.pad(b2, (0, H2P - H2)).reshape(NCHUNK, 1, F)
    w3p = jnp.pad(W3, ((0, H2P - H2), (0, 2048 - 2000)))
    b3p = jnp.pad(b3, (0, 2048 - 2000))[None, :]
    wlp = jnp.pad(Wl, ((0, 2048 - 2000), (0, 128 - 10)))
    blp = jnp.pad(bl, (0, 128 - 10))[None, :]

    srcp = jnp.pad(src, (0, EP - E), constant_values=NP - 1)
    dstp = jnp.pad(dst, (0, EP - E), constant_values=NP - 1)
    src1 = srcp.reshape(NC * NS, E1_NB, E1_BATCH)
    dst1 = dstp.reshape(NC * NS, E1_NB, E1_BATCH)
    src2 = srcp.reshape(NS, E2_NB, E2_BATCH)
    dst2 = dstp.reshape(NS, E2_NB, E2_BATCH)
    zeros_z = jnp.zeros((ROWS_PER_TILE, F), f32)

    xs = _scale_rows(xp, disc)
    parts = _sc1(xs, src1, dst1, zeros_z)
    h1r = _mm1(parts, xs, disc, w1b, b1p)

    ab = jnp.zeros((NP, NP), jnp.bfloat16).at[dst, src].add(
        jnp.ones((), jnp.bfloat16))
    ms_a, _ = _mm2(h1r, w2b, disc, 0)
    qh_a = _sc2(ms_a.reshape(NCHUNK_H * NP, F), src2, dst2, zeros_z)
    ms_b, msb2 = _mm2(h1r, w2b, disc, 1)
    qh_b = _amm(ab, msb2)

    v_a = _vred(qh_a, ms_a, uc, disc, b2r[:NCHUNK_H])
    v_b = _vred(qh_b, ms_b, uc, disc, b2r[NCHUNK_H:])
    v = jnp.concatenate([v_a, v_b]).reshape(1, H2P)
    t = _linear(v, w3p, b3p, 512)
    out = _linear(t, wlp, blp, 128)
    return out[:, :10]
